# Initial kernel scaffold; baseline (speedup 1.0000x reference)
import functools

import jax
import jax.numpy as jnp
from jax import lax
from jax.experimental import pallas as pl
from jax.experimental.pallas import tpu as pltpu
from jax.experimental.pallas import tpu_sc as plsc

N = 50000
E = 800000
D_FEAT = 64
D_OUT = 4

NC = 2
NS = 16
NW = NC * NS
C_PER_W = E // NW
SUB = 5000
N_SUB = C_PER_W // SUB

ROWS_A = 2000
ROWS_B = 20000


def _xwb_body(x_ref, w1_ref, b_ref, out_ref):
    out_ref[...] = (
        jnp.dot(x_ref[...], w1_ref[...], preferred_element_type=jnp.float32)
        + b_ref[...]
    )


def _stage1_xwb(x, w1, b2):
    return pl.pallas_call(
        _xwb_body,
        grid=(N // ROWS_A,),
        in_specs=[
            pl.BlockSpec((ROWS_A, D_FEAT), lambda i: (i, 0)),
            pl.BlockSpec((D_FEAT, D_OUT), lambda i: (0, 0)),
            pl.BlockSpec((1, D_OUT), lambda i: (0, 0)),
        ],
        out_specs=pl.BlockSpec((ROWS_A, D_OUT), lambda i: (i, 0)),
        out_shape=jax.ShapeDtypeStruct((N, D_OUT), jnp.float32),
    )(x, w1, b2)


def _gather_body(col_hbm, xwb_hbm, out_hbm, idx_v, rows_v, sem):
    wid = lax.axis_index("s") * NC + lax.axis_index("c")
    for it in range(N_SUB):
        base = wid * C_PER_W + it * SUB
        pltpu.sync_copy(col_hbm.at[pl.ds(base, SUB)], idx_v)
        pltpu.async_copy(xwb_hbm.at[idx_v], rows_v, sem).wait()
        pltpu.sync_copy(rows_v, out_hbm.at[pl.ds(base, SUB)])


_stage2_gather = functools.partial(
    pl.kernel,
    mesh=plsc.VectorSubcoreMesh(core_axis_name="c", subcore_axis_name="s"),
    out_type=jax.ShapeDtypeStruct((E, D_OUT), jnp.float32),
    scratch_types=[
        pltpu.VMEM((SUB,), jnp.int32),
        pltpu.VMEM((SUB, D_OUT), jnp.float32),
        pltpu.SemaphoreType.DMA,
    ],
)(_gather_body)


def _final_body(gth_ref, ea_ref, w2_ref, out_ref):
    prod = jnp.dot(ea_ref[...], w2_ref[...], preferred_element_type=jnp.float32)
    out_ref[...] = jnp.maximum(gth_ref[...] + prod, 0.0)


def _stage3_final(gth, edge_attr, w2):
    return pl.pallas_call(
        _final_body,
        grid=(E // ROWS_B,),
        in_specs=[
            pl.BlockSpec((ROWS_B, D_OUT), lambda i: (i, 0)),
            pl.BlockSpec((ROWS_B, D_OUT), lambda i: (i, 0)),
            pl.BlockSpec((D_OUT, D_OUT), lambda i: (0, 0)),
        ],
        out_specs=pl.BlockSpec((ROWS_B, D_OUT), lambda i: (i, 0)),
        out_shape=jax.ShapeDtypeStruct((E, D_OUT), jnp.float32),
    )(gth, edge_attr, w2)


@jax.jit
def kernel(x, edge_index, edge_attr, W, b):
    col = edge_index[1].astype(jnp.int32)
    w1 = W[:D_FEAT]
    w2 = W[D_FEAT:]
    b2 = b.reshape(1, D_OUT)
    xwb = _stage1_xwb(x, w1, b2)
    gth = _stage2_gather(col, xwb)
    return _stage3_final(gth, edge_attr, w2)


# speedup vs baseline: 1.2539x; 1.2539x over previous
import functools

import jax
import jax.numpy as jnp
from jax import lax
from jax.experimental import pallas as pl
from jax.experimental.pallas import tpu as pltpu
from jax.experimental.pallas import tpu_sc as plsc

N = 50000
E = 800000
D_FEAT = 64
D_OUT = 4

NC = 2
NS = 16
NW = NC * NS
NR = NW // D_OUT
RANGE = E // NR
SUB = 20000
N_SUB = RANGE // SUB

ROWS_A = 10000
ROWS_B = 5000
EPR = 128 // D_OUT


def _xwbt_body(x_ref, w1_ref, b_ref, out_ref):
    prod = lax.dot_general(
        w1_ref[...], x_ref[...],
        (((0,), (1,)), ((), ())),
        preferred_element_type=jnp.float32,
    )
    out_ref[...] = prod + b_ref[...]


def _stage1_xwbt(x, w1, b2):
    return pl.pallas_call(
        _xwbt_body,
        out_shape=jax.ShapeDtypeStruct((D_OUT, N), jnp.float32),
    )(x, w1, b2)


def _gather_body(col_hbm, tab_hbm, out_hbm, tab_v, col_v, out_v):
    wid = lax.axis_index("s") * NC + lax.axis_index("c")
    j = wid % D_OUT
    r = wid // D_OUT
    pltpu.sync_copy(tab_hbm.at[pl.ds(j * N, N)], tab_v)
    for it in range(N_SUB):
        base = r * RANGE + it * SUB
        pltpu.sync_copy(col_hbm.at[pl.ds(base, SUB)], col_v)

        @plsc.parallel_loop(0, SUB, step=16, unroll=10)
        def _(i):
            idxv = col_v[pl.ds(i, 16)]
            out_v[pl.ds(i, 16)] = plsc.load_gather(tab_v, [idxv])

        pltpu.sync_copy(out_v, out_hbm.at[pl.ds(j * E + base, SUB)])


@functools.cache
def _stage2_gather():
    return pl.kernel(
        _gather_body,
        mesh=plsc.VectorSubcoreMesh(
            core_axis_name="c", subcore_axis_name="s",
            num_cores=NC, num_subcores=NS,
        ),
        out_type=jax.ShapeDtypeStruct((D_OUT * E,), jnp.float32),
        scratch_types=[
            pltpu.VMEM((N,), jnp.float32),
            pltpu.VMEM((SUB,), jnp.int32),
            pltpu.VMEM((SUB,), jnp.float32),
        ],
        compiler_params=pltpu.CompilerParams(needs_layout_passes=False),
    )


def _final_body(g0, g1, g2, g3, ea_ref, sel_ref, w2big_ref, out_ref):
    acc = lax.dot_general(
        ea_ref[...], w2big_ref[...],
        (((1,), (0,)), ((), ())),
        preferred_element_type=jnp.float32,
    )
    for jj, g in enumerate((g0, g1, g2, g3)):
        acc += lax.dot_general(
            g[...], sel_ref[jj],
            (((1,), (0,)), ((), ())),
            preferred_element_type=jnp.float32,
        )
    out_ref[...] = jnp.maximum(acc, 0.0)


def _stage3_final(gcols, ea_flat, sel, w2big):
    rows = E // EPR
    g_spec = pl.BlockSpec((ROWS_B, EPR), lambda i: (i, 0))
    return pl.pallas_call(
        _final_body,
        grid=(rows // ROWS_B,),
        in_specs=[
            g_spec, g_spec, g_spec, g_spec,
            pl.BlockSpec((ROWS_B, 128), lambda i: (i, 0)),
            pl.BlockSpec((D_OUT, EPR, 128), lambda i: (0, 0, 0)),
            pl.BlockSpec((128, 128), lambda i: (0, 0)),
        ],
        out_specs=pl.BlockSpec((ROWS_B, 128), lambda i: (i, 0)),
        out_shape=jax.ShapeDtypeStruct((rows, 128), jnp.float32),
    )(*gcols, ea_flat, sel, w2big)


@jax.jit
def kernel(x, edge_index, edge_attr, W, b):
    col = edge_index[1].astype(jnp.int32)
    w1 = W[:D_FEAT]
    w2 = W[D_FEAT:]
    b2 = b.reshape(D_OUT, 1)

    eye = jnp.eye(EPR, dtype=jnp.float32)
    w2big = jnp.kron(eye, w2)
    sel = jnp.stack([
        jnp.kron(eye, jnp.eye(D_OUT, dtype=jnp.float32)[jj : jj + 1])
        for jj in range(D_OUT)
    ])

    xwbt = _stage1_xwbt(x, w1, b2)
    gflat = _stage2_gather()(col, xwbt.reshape(-1))

    rows = E // EPR
    gcols = [
        gflat[jj * E : (jj + 1) * E].reshape(rows, EPR) for jj in range(D_OUT)
    ]
    out = _stage3_final(gcols, edge_attr.reshape(rows, 128), sel, w2big)
    return out.reshape(E, D_OUT)


# speedup vs baseline: 12.9517x; 10.3292x over previous
import functools

import jax
import jax.numpy as jnp
from jax import lax
from jax.experimental import pallas as pl
from jax.experimental.pallas import tpu as pltpu
from jax.experimental.pallas import tpu_sc as plsc

N = 50000
E = 800000
D_FEAT = 64
D_OUT = 4
G = E // 128

NC = 2
NS = 16
NW = NC * NS
NR = NW // D_OUT
G_PER_W = 784
G_SUB = 49
N_SUB = G_PER_W // G_SUB
E_SUB = G_SUB * 128
LAST_GB = G - G_SUB


def _xwbt_body(x_ref, w1_ref, b_ref, out_ref):
    prod = lax.dot_general(
        w1_ref[...], x_ref[...],
        (((0,), (1,)), ((), ())),
        preferred_element_type=jnp.float32,
    )
    out_ref[...] = prod + b_ref[...]


def _stage1_xwbt(x, w1, b2):
    return pl.pallas_call(
        _xwbt_body,
        out_shape=jax.ShapeDtypeStruct((D_OUT, N), jnp.float32),
    )(x, w1, b2)


def _gather_body(col_hbm, tab_hbm, ea_hbm, w2_hbm, out_hbm,
                 tab_v, col_v, ea_v, out_v, w2_v):
    wid = lax.axis_index("s") * NC + lax.axis_index("c")
    j = wid % D_OUT
    r = wid // D_OUT
    pltpu.sync_copy(tab_hbm.at[pl.ds(j * N, N)], tab_v)
    pltpu.sync_copy(w2_hbm.at[pl.ds(j * 64, 64)], w2_v)
    for it in range(N_SUB):
        gb = jnp.minimum(r * G_PER_W + it * G_SUB, LAST_GB)
        eb = gb * 128
        pltpu.sync_copy(col_hbm.at[pl.ds(eb, E_SUB)], col_v)
        pltpu.sync_copy(ea_hbm.at[pl.ds(gb, G_SUB)], ea_v)

        @plsc.parallel_loop(0, E_SUB // 16, step=1, unroll=8)
        def _(i):
            gl = i // 8
            l16 = (i % 8) * 16
            idxv = col_v[pl.ds(i * 16, 16)]
            acc = plsc.load_gather(tab_v, [idxv])
            for k in range(D_OUT):
                w2k = w2_v[pl.ds(k * 16, 16)]
                eak = ea_v[gl, k, pl.ds(l16, 16)]
                acc = acc + w2k * eak
            out_v[gl, pl.ds(l16, 16)] = jnp.maximum(acc, 0.0)

        pltpu.sync_copy(out_v, out_hbm.at[pl.ds(gb, G_SUB), j])


@functools.cache
def _stage2_gather():
    return pl.kernel(
        _gather_body,
        mesh=plsc.VectorSubcoreMesh(
            core_axis_name="c", subcore_axis_name="s",
            num_cores=NC, num_subcores=NS,
        ),
        out_type=jax.ShapeDtypeStruct((G, D_OUT, 128), jnp.float32),
        scratch_types=[
            pltpu.VMEM((N,), jnp.float32),
            pltpu.VMEM((E_SUB,), jnp.int32),
            pltpu.VMEM((G_SUB, D_OUT, 128), jnp.float32),
            pltpu.VMEM((G_SUB, 128), jnp.float32),
            pltpu.VMEM((64,), jnp.float32),
        ],
        compiler_params=pltpu.CompilerParams(needs_layout_passes=False),
    )


@jax.jit
def kernel(x, edge_index, edge_attr, W, b):
    col = edge_index[1].astype(jnp.int32)
    w1 = W[:D_FEAT]
    w2 = W[D_FEAT:]
    b2 = b.reshape(D_OUT, 1)
    w2rep = jnp.broadcast_to(
        w2.T[:, :, None], (D_OUT, D_OUT, 16)
    ).reshape(-1)

    xwbt = _stage1_xwbt(x, w1, b2)
    ea3 = edge_attr.T.reshape(D_OUT, G, 128).transpose(1, 0, 2)
    out3 = _stage2_gather()(col, xwbt.reshape(-1), ea3, w2rep)
    return out3.transpose(1, 0, 2).reshape(D_OUT, E).T


# speedup vs baseline: 21.8329x; 1.6857x over previous
import functools

import jax
import jax.numpy as jnp
from jax import lax
from jax.experimental import pallas as pl
from jax.experimental.pallas import tpu as pltpu
from jax.experimental.pallas import tpu_sc as plsc

N = 50000
E = 800000
D_FEAT = 64
D_OUT = 4
G = E // 128

NC = 2
NS = 16
NW = NC * NS
NR = NW // D_OUT
G_PER_W = 784
G_SUB = 28
N_SUB = G_PER_W // G_SUB
LAST_GB = G - G_SUB


def _xwbt_body(x_ref, w1_ref, b_ref, out_ref):
    prod = lax.dot_general(
        w1_ref[...], x_ref[...],
        (((0,), (1,)), ((), ())),
        preferred_element_type=jnp.float32,
    )
    out_ref[...] = prod + b_ref[...]


def _stage1_xwbt(x, w1, b2):
    return pl.pallas_call(
        _xwbt_body,
        out_shape=jax.ShapeDtypeStruct((D_OUT, N), jnp.float32),
    )(x, w1, b2)


def _gather_body(ei_hbm, tab_hbm, ea_hbm, w2_hbm, out_hbm,
                 tab_v, w2_v, col_v, ea_v, out_v, sems):
    wid = lax.axis_index("s") * NC + lax.axis_index("c")
    j = wid % D_OUT
    r = wid // D_OUT
    pltpu.sync_copy(tab_hbm.at[pl.ds(j * N, N)], tab_v)
    pltpu.sync_copy(w2_hbm.at[pl.ds(j * 64, 64)], w2_v)

    def start_loads(it):
        buf = it % 2
        gb = jnp.minimum(r * G_PER_W + it * G_SUB, LAST_GB)
        dc = pltpu.async_copy(
            ei_hbm.at[pl.ds(gb, G_SUB), 1], col_v.at[buf], sems.at[buf, 0]
        )
        de = pltpu.async_copy(
            ea_hbm.at[pl.ds(gb, G_SUB)], ea_v.at[buf], sems.at[buf, 1]
        )
        return dc, de, gb

    pending_out = [None, None]
    cur = start_loads(0)
    for it in range(N_SUB):
        buf = it % 2
        nxt = start_loads(it + 1) if it + 1 < N_SUB else None
        cur[0].wait()
        cur[1].wait()
        if pending_out[buf] is not None:
            pending_out[buf].wait()

        cbuf = col_v.at[buf]
        ebuf = ea_v.at[buf]
        obuf = out_v.at[buf]

        @plsc.parallel_loop(0, G_SUB * 8, step=1, unroll=8)
        def _(i):
            gl = i // 8
            l16 = (i % 8) * 16
            idxv = cbuf[gl, pl.ds(l16, 16)]
            acc = plsc.load_gather(tab_v, [idxv])
            for k in range(D_OUT):
                w2k = w2_v[pl.ds(k * 16, 16)]
                eak = ebuf[gl, k, pl.ds(l16, 16)]
                acc = acc + w2k * eak
            obuf[gl, pl.ds(l16, 16)] = jnp.maximum(acc, 0.0)

        do = pltpu.async_copy(
            out_v.at[buf], out_hbm.at[pl.ds(cur[2], G_SUB), j],
            sems.at[buf, 2],
        )
        pending_out[buf] = do
        cur = nxt
    pending_out[0].wait()
    pending_out[1].wait()


@functools.cache
def _stage2_gather():
    return pl.kernel(
        _gather_body,
        mesh=plsc.VectorSubcoreMesh(
            core_axis_name="c", subcore_axis_name="s",
            num_cores=NC, num_subcores=NS,
        ),
        out_type=jax.ShapeDtypeStruct((G, D_OUT, 128), jnp.float32),
        scratch_types=[
            pltpu.VMEM((N,), jnp.float32),
            pltpu.VMEM((64,), jnp.float32),
            pltpu.VMEM((2, G_SUB, 128), jnp.int32),
            pltpu.VMEM((2, G_SUB, D_OUT, 128), jnp.float32),
            pltpu.VMEM((2, G_SUB, 128), jnp.float32),
            pltpu.SemaphoreType.DMA((2, 3)),
        ],
        compiler_params=pltpu.CompilerParams(needs_layout_passes=False),
    )


@jax.jit
def kernel(x, edge_index, edge_attr, W, b):
    w1 = W[:D_FEAT]
    w2 = W[D_FEAT:]
    b2 = b.reshape(D_OUT, 1)
    w2rep = jnp.broadcast_to(
        w2.T[:, :, None], (D_OUT, D_OUT, 16)
    ).reshape(-1)

    xwbt = _stage1_xwbt(x, w1, b2)
    ei3 = edge_index.astype(jnp.int32).reshape(2, G, 128).transpose(1, 0, 2)
    ea3 = edge_attr.T.reshape(D_OUT, G, 128).transpose(1, 0, 2)
    out3 = _stage2_gather()(ei3, xwbt.reshape(-1), ea3, w2rep)
    return out3.transpose(1, 0, 2).reshape(D_OUT, E).T


# speedup vs baseline: 29.9454x; 1.3716x over previous
import functools

import jax
import jax.numpy as jnp
from jax import lax
from jax.experimental import pallas as pl
from jax.experimental.pallas import tpu as pltpu
from jax.experimental.pallas import tpu_sc as plsc

N = 50000
E = 800000
D_FEAT = 64
D_OUT = 4
G = E // 128

NC = 2
NS = 16
NW = NC * NS
NP = 2
NRANGE = NW // NP
G_PER_W = 392
G_SUB = 14
N_SUB = G_PER_W // G_SUB
LAST_GB = G - G_SUB
K_STEPS = 8


def _xwbt_body(xt_ref, w1_ref, b_ref, out_ref):
    i = pl.program_id(0)
    prod = lax.dot_general(
        w1_ref[...], xt_ref[...],
        (((0,), (0,)), ((), ())),
        preferred_element_type=jnp.float32,
    )

    @pl.when(i == 0)
    def _():
        out_ref[...] = prod + b_ref[...]

    @pl.when(i != 0)
    def _():
        out_ref[...] += prod


def _stage1_xwbt(xt, w1, b2):
    kc = D_FEAT // K_STEPS
    return pl.pallas_call(
        _xwbt_body,
        grid=(K_STEPS,),
        in_specs=[
            pl.BlockSpec((kc, N), lambda i: (i, 0)),
            pl.BlockSpec((kc, D_OUT), lambda i: (i, 0)),
            pl.BlockSpec((D_OUT, 1), lambda i: (0, 0)),
        ],
        out_specs=pl.BlockSpec((D_OUT, N), lambda i: (0, 0)),
        out_shape=jax.ShapeDtypeStruct((D_OUT, N), jnp.float32),
    )(xt, w1, b2)


def _gather_body(ei_hbm, tab_hbm, ea_hbm, w2_hbm, out_hbm,
                 tab_v, w2_v, col_v, ea_v, out_v, sems):
    wid = lax.axis_index("s") * NC + lax.axis_index("c")
    p = wid % NP
    r = wid // NP
    pltpu.sync_copy(tab_hbm.at[pl.ds(p * (2 * N), 2 * N)], tab_v)
    pltpu.sync_copy(w2_hbm.at[pl.ds(p * 128, 128)], w2_v)
    w2s = [w2_v[pl.ds(k * 16, 16)] for k in range(8)]

    def start_loads(it):
        buf = it % 2
        gb = jnp.minimum(r * G_PER_W + it * G_SUB, LAST_GB)
        dc = pltpu.async_copy(
            ei_hbm.at[pl.ds(gb, G_SUB), 1], col_v.at[buf], sems.at[buf, 0]
        )
        de = pltpu.async_copy(
            ea_hbm.at[pl.ds(gb, G_SUB)], ea_v.at[buf], sems.at[buf, 1]
        )
        return dc, de, gb

    pending_out = [None, None]
    cur = start_loads(0)
    for it in range(N_SUB):
        buf = it % 2
        nxt = start_loads(it + 1) if it + 1 < N_SUB else None
        cur[0].wait()
        cur[1].wait()
        if pending_out[buf] is not None:
            pending_out[buf].wait()

        cbuf = col_v.at[buf]
        ebuf = ea_v.at[buf]
        obuf = out_v.at[buf]

        @plsc.parallel_loop(0, G_SUB * 8, step=1, unroll=8)
        def _(i):
            gl = i // 8
            l16 = (i % 8) * 16
            idxv = cbuf[gl, pl.ds(l16, 16)]
            acc0 = plsc.load_gather(tab_v, [idxv])
            acc1 = plsc.load_gather(tab_v, [idxv + N])
            for k in range(D_OUT):
                eak = ebuf[gl, k, pl.ds(l16, 16)]
                acc0 = acc0 + w2s[k] * eak
                acc1 = acc1 + w2s[4 + k] * eak
            obuf[gl, 0, pl.ds(l16, 16)] = jnp.maximum(acc0, 0.0)
            obuf[gl, 1, pl.ds(l16, 16)] = jnp.maximum(acc1, 0.0)

        do = pltpu.async_copy(
            out_v.at[buf],
            out_hbm.at[pl.ds(cur[2], G_SUB), pl.ds(2 * p, 2)],
            sems.at[buf, 2],
        )
        pending_out[buf] = do
        cur = nxt
    pending_out[0].wait()
    pending_out[1].wait()


@functools.cache
def _stage2_gather():
    return pl.kernel(
        _gather_body,
        mesh=plsc.VectorSubcoreMesh(
            core_axis_name="c", subcore_axis_name="s",
            num_cores=NC, num_subcores=NS,
        ),
        out_type=jax.ShapeDtypeStruct((G, D_OUT, 128), jnp.float32),
        scratch_types=[
            pltpu.VMEM((2 * N,), jnp.float32),
            pltpu.VMEM((128,), jnp.float32),
            pltpu.VMEM((2, G_SUB, 128), jnp.int32),
            pltpu.VMEM((2, G_SUB, D_OUT, 128), jnp.float32),
            pltpu.VMEM((2, G_SUB, 2, 128), jnp.float32),
            pltpu.SemaphoreType.DMA((2, 3)),
        ],
        compiler_params=pltpu.CompilerParams(needs_layout_passes=False),
    )


@jax.jit
def kernel(x, edge_index, edge_attr, W, b):
    w1 = W[:D_FEAT]
    w2 = W[D_FEAT:]
    b2 = b.reshape(D_OUT, 1)
    w2rep = jnp.broadcast_to(
        w2.T[:, :, None], (D_OUT, D_OUT, 16)
    ).reshape(-1)

    xwbt = _stage1_xwbt(x.T, w1, b2)
    ei3 = edge_index.astype(jnp.int32).reshape(2, G, 128).transpose(1, 0, 2)
    ea3 = edge_attr.T.reshape(D_OUT, G, 128).transpose(1, 0, 2)
    out3 = _stage2_gather()(ei3, xwbt.reshape(-1), ea3, w2rep)
    return out3.transpose(1, 0, 2).reshape(D_OUT, E).T


# speedup vs baseline: 35.0127x; 1.1692x over previous
import functools

import jax
import jax.numpy as jnp
from jax import lax
from jax.experimental import pallas as pl
from jax.experimental.pallas import tpu as pltpu
from jax.experimental.pallas import tpu_sc as plsc

N = 50000
E = 800000
D_FEAT = 64
D_OUT = 4
G = E // 128

NC = 2
NS = 16
NW = NC * NS
NP = 2
NRANGE = NW // NP
G_PER_W = 392
G_SUB = 28
N_SUB = G_PER_W // G_SUB
LAST_GB = G - G_SUB
K_STEPS = 8


def _pack_pair(hi, lo):
    hb = lax.bitcast_convert_type(
        hi.astype(jnp.bfloat16), jnp.uint16
    ).astype(jnp.uint32)
    lb = lax.bitcast_convert_type(
        lo.astype(jnp.bfloat16), jnp.uint16
    ).astype(jnp.uint32)
    return lax.bitcast_convert_type((hb << 16) | lb, jnp.float32)


def _xwbt_body(xt_ref, w1_ref, b_ref, out_ref, acc_ref):
    i = pl.program_id(0)
    prod = lax.dot_general(
        w1_ref[...], xt_ref[...],
        (((0,), (0,)), ((), ())),
        preferred_element_type=jnp.float32,
    )

    @pl.when(i == 0)
    def _():
        acc_ref[...] = prod + b_ref[...]

    @pl.when(i != 0)
    def _():
        acc_ref[...] += prod

    @pl.when(i == K_STEPS - 1)
    def _():
        a = acc_ref[...]
        out_ref[...] = jnp.concatenate(
            [_pack_pair(a[0:1], a[1:2]), _pack_pair(a[2:3], a[3:4])], axis=0
        )


def _stage1_xwbt(xt, w1, b2):
    kc = D_FEAT // K_STEPS
    return pl.pallas_call(
        _xwbt_body,
        grid=(K_STEPS,),
        in_specs=[
            pl.BlockSpec((kc, N), lambda i: (i, 0)),
            pl.BlockSpec((kc, D_OUT), lambda i: (i, 0)),
            pl.BlockSpec((D_OUT, 1), lambda i: (0, 0)),
        ],
        out_specs=pl.BlockSpec((NP, N), lambda i: (0, 0)),
        out_shape=jax.ShapeDtypeStruct((NP, N), jnp.float32),
        scratch_shapes=[pltpu.VMEM((D_OUT, N), jnp.float32)],
    )(xt, w1, b2)


def _gather_body(ei_hbm, tab_hbm, ea_hbm, w2_hbm, out_hbm,
                 tab_v, w2_v, col_v, ea_v, out_v, sems):
    wid = lax.axis_index("s") * NC + lax.axis_index("c")
    p = wid % NP
    r = wid // NP
    pltpu.sync_copy(tab_hbm.at[pl.ds(p * N, N)], tab_v)
    pltpu.sync_copy(w2_hbm.at[pl.ds(p * 128, 128)], w2_v)
    w2s = [w2_v[pl.ds(k * 16, 16)] for k in range(8)]

    def start_loads(it):
        buf = it % 2
        gb = jnp.minimum(r * G_PER_W + it * G_SUB, LAST_GB)
        dc = pltpu.async_copy(
            ei_hbm.at[pl.ds(gb, G_SUB), 1], col_v.at[buf], sems.at[buf, 0]
        )
        de = pltpu.async_copy(
            ea_hbm.at[pl.ds(gb, G_SUB)], ea_v.at[buf], sems.at[buf, 1]
        )
        return dc, de, gb

    pending_out = [None, None]
    cur = start_loads(0)
    for it in range(N_SUB):
        buf = it % 2
        nxt = start_loads(it + 1) if it + 1 < N_SUB else None
        cur[0].wait()
        cur[1].wait()
        if pending_out[buf] is not None:
            pending_out[buf].wait()

        cbuf = col_v.at[buf]
        ebuf = ea_v.at[buf]
        obuf = out_v.at[buf]

        @plsc.parallel_loop(0, G_SUB * 8, step=1, unroll=8)
        def _(i):
            gl = i // 8
            l16 = (i % 8) * 16
            idxv = cbuf[gl, pl.ds(l16, 16)]
            gv = plsc.load_gather(tab_v, [idxv])
            lo, hi = plsc.unpack(
                plsc.bitcast(gv, jnp.bfloat16),
                format=plsc.PackFormat.INTERLEAVED,
            )
            acc0 = hi
            acc1 = lo
            for k in range(D_OUT):
                eak = ebuf[gl, k, pl.ds(l16, 16)]
                acc0 = acc0 + w2s[k] * eak
                acc1 = acc1 + w2s[4 + k] * eak
            obuf[gl, 0, pl.ds(l16, 16)] = jnp.maximum(acc0, 0.0)
            obuf[gl, 1, pl.ds(l16, 16)] = jnp.maximum(acc1, 0.0)

        do = pltpu.async_copy(
            out_v.at[buf],
            out_hbm.at[pl.ds(cur[2], G_SUB), pl.ds(2 * p, 2)],
            sems.at[buf, 2],
        )
        pending_out[buf] = do
        cur = nxt
    pending_out[0].wait()
    pending_out[1].wait()


@functools.cache
def _stage2_gather():
    return pl.kernel(
        _gather_body,
        mesh=plsc.VectorSubcoreMesh(
            core_axis_name="c", subcore_axis_name="s",
            num_cores=NC, num_subcores=NS,
        ),
        out_type=jax.ShapeDtypeStruct((G, D_OUT, 128), jnp.float32),
        scratch_types=[
            pltpu.VMEM((N,), jnp.float32),
            pltpu.VMEM((128,), jnp.float32),
            pltpu.VMEM((2, G_SUB, 128), jnp.int32),
            pltpu.VMEM((2, G_SUB, D_OUT, 128), jnp.float32),
            pltpu.VMEM((2, G_SUB, 2, 128), jnp.float32),
            pltpu.SemaphoreType.DMA((2, 3)),
        ],
        compiler_params=pltpu.CompilerParams(needs_layout_passes=False),
    )


@jax.jit
def kernel(x, edge_index, edge_attr, W, b):
    w1 = W[:D_FEAT]
    w2 = W[D_FEAT:]
    b2 = b.reshape(D_OUT, 1)
    w2rep = jnp.broadcast_to(
        w2.T[:, :, None], (D_OUT, D_OUT, 16)
    ).reshape(-1)

    packed = _stage1_xwbt(x.T, w1, b2)
    ei3 = edge_index.astype(jnp.int32).reshape(2, G, 128).transpose(1, 0, 2)
    ea3 = edge_attr.T.reshape(D_OUT, G, 128).transpose(1, 0, 2)
    out3 = _stage2_gather()(ei3, packed.reshape(-1), ea3, w2rep)
    return out3.transpose(1, 0, 2).reshape(D_OUT, E).T


# speedup vs baseline: 36.6232x; 1.0460x over previous
import functools

import jax
import jax.numpy as jnp
from jax import lax
from jax.experimental import pallas as pl
from jax.experimental.pallas import tpu as pltpu
from jax.experimental.pallas import tpu_sc as plsc

N = 50000
E = 800000
D_FEAT = 64
D_OUT = 4
G = E // 128

NC = 2
NS = 16
NW = NC * NS
NP = 2
NRANGE = NW // NP
G_PER_W = 392
G_SUB = 28
N_SUB = G_PER_W // G_SUB
LAST_GB = G - G_SUB
K_STEPS = 4


def _pack_pair(hi, lo):
    hb = lax.bitcast_convert_type(
        hi.astype(jnp.bfloat16), jnp.uint16
    ).astype(jnp.uint32)
    lb = lax.bitcast_convert_type(
        lo.astype(jnp.bfloat16), jnp.uint16
    ).astype(jnp.uint32)
    return lax.bitcast_convert_type((hb << 16) | lb, jnp.float32)


def _xwbt_body(xt_ref, w1_ref, b_ref, out_ref, acc_ref):
    i = pl.program_id(0)
    prod = lax.dot_general(
        w1_ref[...], xt_ref[...],
        (((0,), (0,)), ((), ())),
        preferred_element_type=jnp.float32,
    )

    @pl.when(i == 0)
    def _():
        acc_ref[...] = prod + b_ref[...]

    @pl.when(i != 0)
    def _():
        acc_ref[...] += prod

    @pl.when(i == K_STEPS - 1)
    def _():
        a = acc_ref[...]
        out_ref[...] = jnp.concatenate(
            [_pack_pair(a[0:1], a[1:2]), _pack_pair(a[2:3], a[3:4])], axis=0
        )


def _stage1_xwbt(xt, w1, b2):
    kc = D_FEAT // K_STEPS
    return pl.pallas_call(
        _xwbt_body,
        grid=(K_STEPS,),
        in_specs=[
            pl.BlockSpec((kc, N), lambda i: (i, 0)),
            pl.BlockSpec((kc, D_OUT), lambda i: (i, 0)),
            pl.BlockSpec((D_OUT, 1), lambda i: (0, 0)),
        ],
        out_specs=pl.BlockSpec((NP, N), lambda i: (0, 0)),
        out_shape=jax.ShapeDtypeStruct((NP, N), jnp.float32),
        scratch_shapes=[pltpu.VMEM((D_OUT, N), jnp.float32)],
    )(xt, w1, b2)


def _gather_body(ei_hbm, tab_hbm, ea_hbm, w2_hbm, out_hbm,
                 tab_v, w2_v, col_v, ea_v, out_v, sems):
    wid = lax.axis_index("s") * NC + lax.axis_index("c")
    p = wid % NP
    r = wid // NP
    pltpu.sync_copy(tab_hbm.at[pl.ds(p * N, N)], tab_v)
    pltpu.sync_copy(w2_hbm.at[pl.ds(p * 128, 128)], w2_v)
    w2s = [w2_v[pl.ds(k * 16, 16)] for k in range(8)]

    def start_loads(it):
        buf = it % 2
        gb = jnp.minimum(r * G_PER_W + it * G_SUB, LAST_GB)
        dc = pltpu.async_copy(
            ei_hbm.at[pl.ds(gb, G_SUB), 1], col_v.at[buf], sems.at[buf, 0]
        )
        de = pltpu.async_copy(
            ea_hbm.at[pl.ds(gb, G_SUB)], ea_v.at[buf], sems.at[buf, 1]
        )
        return dc, de, gb

    pending_out = [None, None]
    cur = start_loads(0)
    for it in range(N_SUB):
        buf = it % 2
        nxt = start_loads(it + 1) if it + 1 < N_SUB else None
        cur[0].wait()
        cur[1].wait()
        if pending_out[buf] is not None:
            pending_out[buf].wait()

        cbuf = col_v.at[buf]
        ebuf = ea_v.at[buf]
        obuf = out_v.at[buf]

        @plsc.parallel_loop(0, G_SUB * 8, step=1, unroll=8)
        def _(i):
            gl = i // 8
            l16 = (i % 8) * 16
            idxv = cbuf[gl, pl.ds(l16, 16)]
            gv = plsc.load_gather(tab_v, [idxv])
            lo, hi = plsc.unpack(
                plsc.bitcast(gv, jnp.bfloat16),
                format=plsc.PackFormat.INTERLEAVED,
            )
            acc0 = hi
            acc1 = lo
            for k in range(D_OUT):
                eak = ebuf[gl, k, pl.ds(l16, 16)]
                acc0 = acc0 + w2s[k] * eak
                acc1 = acc1 + w2s[4 + k] * eak
            obuf[gl, 0, pl.ds(l16, 16)] = jnp.maximum(acc0, 0.0)
            obuf[gl, 1, pl.ds(l16, 16)] = jnp.maximum(acc1, 0.0)

        do = pltpu.async_copy(
            out_v.at[buf],
            out_hbm.at[pl.ds(cur[2], G_SUB), pl.ds(2 * p, 2)],
            sems.at[buf, 2],
        )
        pending_out[buf] = do
        cur = nxt
    pending_out[0].wait()
    pending_out[1].wait()


@functools.cache
def _stage2_gather():
    return pl.kernel(
        _gather_body,
        mesh=plsc.VectorSubcoreMesh(
            core_axis_name="c", subcore_axis_name="s",
            num_cores=NC, num_subcores=NS,
        ),
        out_type=jax.ShapeDtypeStruct((G, D_OUT, 128), jnp.float32),
        scratch_types=[
            pltpu.VMEM((N,), jnp.float32),
            pltpu.VMEM((128,), jnp.float32),
            pltpu.VMEM((2, G_SUB, 128), jnp.int32),
            pltpu.VMEM((2, G_SUB, D_OUT, 128), jnp.float32),
            pltpu.VMEM((2, G_SUB, 2, 128), jnp.float32),
            pltpu.SemaphoreType.DMA((2, 3)),
        ],
        compiler_params=pltpu.CompilerParams(needs_layout_passes=False),
    )


@jax.jit
def kernel(x, edge_index, edge_attr, W, b):
    w1 = W[:D_FEAT]
    w2 = W[D_FEAT:]
    b2 = b.reshape(D_OUT, 1)
    w2rep = jnp.broadcast_to(
        w2.T[:, :, None], (D_OUT, D_OUT, 16)
    ).reshape(-1)

    packed = _stage1_xwbt(x.T, w1, b2)
    ei3 = edge_index.astype(jnp.int32).reshape(2, G, 128).transpose(1, 0, 2)
    ea3 = edge_attr.T.reshape(D_OUT, G, 128).transpose(1, 0, 2)
    out3 = _stage2_gather()(ei3, packed.reshape(-1), ea3, w2rep)
    return out3.transpose(1, 0, 2).reshape(D_OUT, E).T
